# baseline (speedup 1.0000x reference)
import functools

import jax
import jax.numpy as jnp
from jax import lax
from jax.experimental import pallas as pl
from jax.experimental.pallas import tpu as pltpu
from jax.experimental.pallas import tpu_sc as plsc

NUM_ENTITIES = 1000000
NUM_RELATIONS = 1000
EMBED_DIM = 32
BATCH = 16384
PACK = 128 // EMBED_DIM
PACK_SHIFT = 2

NC, NS, L = 2, 16, 16
NW = NC * NS
B_PER_W = BATCH // NW
CHUNK = 256
NCHUNK = B_PER_W // CHUNK
GPC = CHUNK // L

_mesh = plsc.VectorSubcoreMesh(core_axis_name="c", subcore_axis_name="s")


@functools.partial(
    pl.kernel,
    out_type=jax.ShapeDtypeStruct((BATCH,), jnp.float32),
    mesh=_mesh,
    scratch_types=[
        pltpu.VMEM((B_PER_W,), jnp.int32),
        pltpu.VMEM((B_PER_W,), jnp.int32),
        pltpu.VMEM((B_PER_W,), jnp.int32),
        [pltpu.VMEM((CHUNK,), jnp.int32) for _ in range(NCHUNK)],
        [pltpu.VMEM((CHUNK,), jnp.int32) for _ in range(NCHUNK)],
        pltpu.VMEM((CHUNK, 128), jnp.float32),
        pltpu.VMEM((CHUNK, 128), jnp.float32),
        pltpu.VMEM((NUM_RELATIONS // PACK, 128), jnp.float32),
        pltpu.VMEM((B_PER_W,), jnp.float32),
        pltpu.SemaphoreType.DMA,
        pltpu.SemaphoreType.DMA,
    ],
    compiler_params=pltpu.CompilerParams(needs_layout_passes=False),
)
def _score_kernel(head_hbm, rel_hbm, tail_hbm, ent_hbm, reltab_hbm, out_hbm,
                  hi, ri, ti, hi4, ti4, h4, t4, relv, out_v, sem, rsem):
    wid = lax.axis_index("s") * NC + lax.axis_index("c")
    base = wid * B_PER_W

    pltpu.sync_copy(head_hbm.at[pl.ds(base, B_PER_W)], hi)
    pltpu.sync_copy(rel_hbm.at[pl.ds(base, B_PER_W)], ri)
    pltpu.sync_copy(tail_hbm.at[pl.ds(base, B_PER_W)], ti)
    cp_rel = pltpu.async_copy(reltab_hbm, relv, rsem)

    for c0 in range(NCHUNK):
        for v in range(CHUNK // L):
            sl = pl.ds(c0 * CHUNK + v * L, L)
            dl = pl.ds(v * L, L)
            hi4[c0][dl] = hi[sl] >> PACK_SHIFT
            ti4[c0][dl] = ti[sl] >> PACK_SHIFT

    lane = lax.iota(jnp.int32, L)

    def compute_group(c, g):
        off = c * CHUNK + g * L
        idh = hi[pl.ds(off, L)]
        idt = ti[pl.ds(off, L)]
        idr = ri[pl.ds(off, L)]
        ch0 = (idh & (PACK - 1)) << 5
        ct0 = (idt & (PACK - 1)) << 5
        cr0 = (idr & (PACK - 1)) << 5
        rr = idr >> PACK_SHIFT
        rows = g * L + lane
        acc = jnp.zeros((L,), jnp.float32)
        for j in range(EMBED_DIM):
            h = plsc.load_gather(h4, [rows, ch0 + j])
            t = plsc.load_gather(t4, [rows, ct0 + j])
            r = plsc.load_gather(relv, [rr, cr0 + j])
            d = (h + r) - t
            acc = acc + d * d
        x = acc + jnp.float32(1e-12)
        xi = plsc.bitcast(x, jnp.int32)
        yi = jnp.int32(0x5F3759DF) - (xi >> 1)
        y = plsc.bitcast(yi, jnp.float32)
        half_x = jnp.float32(0.5) * x
        for _ in range(3):
            y = y * (jnp.float32(1.5) - half_x * y * y)
        out_v[pl.ds(off, L)] = -(x * y)

    for c in range(NCHUNK):
        cp_h = pltpu.async_copy(ent_hbm.at[hi4[c]], h4, sem)
        cp_t = pltpu.async_copy(ent_hbm.at[ti4[c]], t4, sem)
        cp_h.wait()
        cp_t.wait()
        if c == 0:
            cp_rel.wait()

        def group_body(g, _):
            compute_group(c, g)
            return 0

        lax.fori_loop(0, GPC, group_body, 0)

    pltpu.sync_copy(out_v, out_hbm.at[pl.ds(base, B_PER_W)])


def kernel(head_ids, relation_ids, tail_ids, entity_table, relation_table):
    return _score_kernel(
        head_ids.astype(jnp.int32),
        relation_ids.astype(jnp.int32),
        tail_ids.astype(jnp.int32),
        entity_table.reshape(NUM_ENTITIES // PACK, 128),
        relation_table.reshape(NUM_RELATIONS // PACK, 128),
    )
